# Initial kernel scaffold; baseline (speedup 1.0000x reference)
import functools

import jax
import jax.numpy as jnp
from jax import lax
from jax.experimental import pallas as pl
from jax.experimental.pallas import tpu as pltpu
from jax.experimental.pallas import tpu_sc as plsc

N = 10000
E = 320000
D_IN = 128
D_HID = 128
D_OUT = 40
D_PAD = 64

CH = 125
NROWS = E // CH
NW = 32
ROWS_PER_W = NROWS // NW
NODES_PER_S = N // 16


def _sc_aggregate(d, h, srcm, dstm):
    mesh = plsc.VectorSubcoreMesh(core_axis_name="c", subcore_axis_name="s")

    @functools.partial(
        pl.kernel,
        out_type=jax.ShapeDtypeStruct((2, N, d), jnp.float32),
        mesh=mesh,
        scratch_types=[
            pltpu.VMEM((ROWS_PER_W, CH), jnp.int32),
            pltpu.VMEM((ROWS_PER_W, CH), jnp.int32),
            pltpu.VMEM((CH, d), jnp.float32),
            pltpu.VMEM_SHARED((N, d), jnp.float32),
        ],
    )
    def agg_kernel(h_hbm, src_hbm, dst_hbm, out_hbm, sidx, didx, rows, acc):
        cid = lax.axis_index("c")
        sid = lax.axis_index("s")
        wid = sid * 2 + cid
        row0 = wid * ROWS_PER_W

        pltpu.sync_copy(src_hbm.at[pl.ds(row0, ROWS_PER_W)], sidx)
        pltpu.sync_copy(dst_hbm.at[pl.ds(row0, ROWS_PER_W)], didx)

        zv = jnp.zeros((16,), jnp.float32)

        @pl.loop(0, CH)
        def _(i):
            @pl.loop(0, d, step=16)
            def _(j):
                rows[i, pl.ds(j, 16)] = zv

        @pl.loop(0, NODES_PER_S // CH)
        def _(k):
            pltpu.sync_copy(rows, acc.at[pl.ds(sid * NODES_PER_S + k * CH, CH)])

        plsc.subcore_barrier()

        @pl.loop(0, ROWS_PER_W)
        def _(j):
            pltpu.sync_copy(h_hbm.at[sidx.at[j]], rows)
            pltpu.sync_copy(rows, acc.at[didx.at[j]], add=True)

        plsc.subcore_barrier()

        pltpu.sync_copy(acc.at[pl.ds(sid * NODES_PER_S, NODES_PER_S)],
                        out_hbm.at[cid].at[pl.ds(sid * NODES_PER_S, NODES_PER_S)])

    return agg_kernel(h, srcm, dstm)


def _mm_body(x_ref, w_ref, o_ref):
    o_ref[...] = jnp.dot(x_ref[...], w_ref[...],
                         preferred_element_type=jnp.float32)


def _tc_matmul(x, w):
    return pl.pallas_call(
        _mm_body,
        out_shape=jax.ShapeDtypeStruct((x.shape[0], w.shape[1]), jnp.float32),
    )(x, w)


def _mid_body(a_ref, b_ref, g_ref, be_ref, w_ref, o_ref):
    h = a_ref[0] + a_ref[1] + b_ref[...]
    mu = jnp.mean(h, axis=0, keepdims=True)
    var = jnp.mean((h - mu) ** 2, axis=0, keepdims=True)
    hn = (h - mu) / jnp.sqrt(var + 1e-5) * g_ref[...] + be_ref[...]
    hn = jnp.maximum(hn, 0.0)
    o_ref[...] = jnp.dot(hn, w_ref[...], preferred_element_type=jnp.float32)


def _tc_mid(a, b, g, be, w):
    return pl.pallas_call(
        _mid_body,
        out_shape=jax.ShapeDtypeStruct((N, w.shape[1]), jnp.float32),
    )(a, b.reshape(1, -1), g.reshape(1, -1), be.reshape(1, -1), w)


def _fin_body(a_ref, b_ref, o_ref):
    h = a_ref[0] + a_ref[1] + b_ref[...]
    col = lax.broadcasted_iota(jnp.int32, h.shape, 1)
    valid = col < D_OUT
    m = jnp.max(jnp.where(valid, h, -jnp.inf), axis=1, keepdims=True)
    ex = jnp.where(valid, jnp.exp(h - m), 0.0)
    lse = m + jnp.log(jnp.sum(ex, axis=1, keepdims=True))
    o_ref[...] = h - lse


def _tc_final(a, b):
    return pl.pallas_call(
        _fin_body,
        out_shape=jax.ShapeDtypeStruct((N, D_PAD), jnp.float32),
    )(a, b.reshape(1, -1))


def kernel(x, edge_index, W0, b0, g0, be0, W1, b1, g1, be1, W2, b2):
    srcm = edge_index[0].reshape(NROWS, CH)
    dstm = edge_index[1].reshape(NROWS, CH)
    W2p = jnp.pad(W2, ((0, 0), (0, D_PAD - D_OUT)))
    b2p = jnp.pad(b2, (0, D_PAD - D_OUT))

    h0 = _tc_matmul(x, W0)
    a0 = _sc_aggregate(D_HID, h0, srcm, dstm)
    h1 = _tc_mid(a0, b0, g0, be0, W1)
    a1 = _sc_aggregate(D_HID, h1, srcm, dstm)
    h2 = _tc_mid(a1, b1, g1, be1, W2p)
    a2 = _sc_aggregate(D_PAD, h2, srcm, dstm)
    out = _tc_final(a2, b2p)
    return out[:, :D_OUT]


# speedup vs baseline: 6.7544x; 6.7544x over previous
import functools

import jax
import jax.numpy as jnp
from jax import lax
from jax.experimental import pallas as pl
from jax.experimental.pallas import tpu as pltpu
from jax.experimental.pallas import tpu_sc as plsc

N = 10000
E = 320000
D_IN = 128
D_HID = 128
D_OUT = 40
D_PAD = 64

CH = 125
NROWS = E // CH
ROWS_PER_SUB = NROWS // 16
NP = 10240
NODES_PER_S = NP // 16
ZCH = 128


def _sc_aggregate(dh, h2, srcm, dstm):
    mesh = plsc.VectorSubcoreMesh(core_axis_name="c", subcore_axis_name="s")

    @functools.partial(
        pl.kernel,
        out_type=jax.ShapeDtypeStruct((2, NP, dh), jnp.float32),
        mesh=mesh,
        compiler_params=pltpu.CompilerParams(use_tc_tiling_on_sc=False),
        scratch_types=[
            pltpu.VMEM((ROWS_PER_SUB, CH), jnp.int32),
            pltpu.VMEM((ROWS_PER_SUB, CH), jnp.int32),
            pltpu.VMEM((CH, dh), jnp.float32),
            pltpu.VMEM((ZCH, dh), jnp.float32),
            pltpu.VMEM_SHARED((NP, dh), jnp.float32),
        ],
    )
    def agg_kernel(h_hbm, src_hbm, dst_hbm, out_hbm, sidx, didx, rows, zbuf, acc):
        cid = lax.axis_index("c")
        sid = lax.axis_index("s")
        row0 = sid * ROWS_PER_SUB

        pltpu.sync_copy(src_hbm.at[pl.ds(row0, ROWS_PER_SUB)], sidx)
        pltpu.sync_copy(dst_hbm.at[pl.ds(row0, ROWS_PER_SUB)], didx)

        zv = jnp.zeros((16,), jnp.float32)

        @pl.loop(0, ZCH)
        def _(i):
            @pl.loop(0, dh, step=16)
            def _(j):
                zbuf[i, pl.ds(j, 16)] = zv

        @pl.loop(0, NODES_PER_S // ZCH)
        def _(k):
            pltpu.sync_copy(zbuf, acc.at[pl.ds(sid * NODES_PER_S + k * ZCH, ZCH)])

        plsc.subcore_barrier()

        @pl.loop(0, ROWS_PER_SUB)
        def _(j):
            pltpu.sync_copy(h_hbm.at[cid].at[sidx.at[j]], rows)
            pltpu.sync_copy(rows, acc.at[didx.at[j]], add=True)

        plsc.subcore_barrier()

        pltpu.sync_copy(acc.at[pl.ds(sid * NODES_PER_S, NODES_PER_S)],
                        out_hbm.at[cid].at[pl.ds(sid * NODES_PER_S, NODES_PER_S)])

    return agg_kernel(h2, srcm, dstm)


def _split_halves(r):
    dh = r.shape[1] // 2
    return jnp.stack([r[:, :dh], r[:, dh:]])


def _mm_body(x_ref, w_ref, o_ref):
    r = jnp.dot(x_ref[...], w_ref[...], preferred_element_type=jnp.float32)
    o_ref[...] = _split_halves(r)


def _tc_matmul(x, w):
    return pl.pallas_call(
        _mm_body,
        out_shape=jax.ShapeDtypeStruct((2, x.shape[0], w.shape[1] // 2),
                                       jnp.float32),
    )(x, w)


def _mid_body(a_ref, b_ref, g_ref, be_ref, w_ref, o_ref):
    h = jnp.concatenate([a_ref[0, :N], a_ref[1, :N]], axis=1) + b_ref[...]
    mu = jnp.mean(h, axis=0, keepdims=True)
    var = jnp.mean((h - mu) ** 2, axis=0, keepdims=True)
    hn = (h - mu) / jnp.sqrt(var + 1e-5) * g_ref[...] + be_ref[...]
    hn = jnp.maximum(hn, 0.0)
    r = jnp.dot(hn, w_ref[...], preferred_element_type=jnp.float32)
    o_ref[...] = _split_halves(r)


def _tc_mid(a, b, g, be, w):
    return pl.pallas_call(
        _mid_body,
        out_shape=jax.ShapeDtypeStruct((2, N, w.shape[1] // 2), jnp.float32),
    )(a, b.reshape(1, -1), g.reshape(1, -1), be.reshape(1, -1), w)


def _fin_body(a_ref, b_ref, o_ref):
    h = jnp.concatenate([a_ref[0, :N], a_ref[1, :N]], axis=1) + b_ref[...]
    col = lax.broadcasted_iota(jnp.int32, h.shape, 1)
    valid = col < D_OUT
    m = jnp.max(jnp.where(valid, h, -jnp.inf), axis=1, keepdims=True)
    ex = jnp.where(valid, jnp.exp(h - m), 0.0)
    lse = m + jnp.log(jnp.sum(ex, axis=1, keepdims=True))
    o_ref[...] = h - lse


def _tc_final(a, b):
    return pl.pallas_call(
        _fin_body,
        out_shape=jax.ShapeDtypeStruct((N, D_PAD), jnp.float32),
    )(a, b.reshape(1, -1))


def kernel(x, edge_index, W0, b0, g0, be0, W1, b1, g1, be1, W2, b2):
    srcm = edge_index[0].reshape(NROWS, CH)
    dstm = edge_index[1].reshape(NROWS, CH)
    W2p = jnp.pad(W2, ((0, 0), (0, D_PAD - D_OUT)))
    b2p = jnp.pad(b2, (0, D_PAD - D_OUT))

    h0 = _tc_matmul(x, W0)
    a0 = _sc_aggregate(64, h0, srcm, dstm)
    h1 = _tc_mid(a0, b0, g0, be0, W1)
    a1 = _sc_aggregate(64, h1, srcm, dstm)
    h2 = _tc_mid(a1, b1, g1, be1, W2p)
    a2 = _sc_aggregate(32, h2, srcm, dstm)
    out = _tc_final(a2, b2p)
    return out[:, :D_OUT]


# speedup vs baseline: 8.3392x; 1.2346x over previous
import functools

import jax
import jax.numpy as jnp
from jax import lax
from jax.experimental import pallas as pl
from jax.experimental.pallas import tpu as pltpu
from jax.experimental.pallas import tpu_sc as plsc

N = 10000
E = 320000
D_IN = 128
D_HID = 128
D_OUT = 40
D_PAD = 64

CH = 125
NROWS = E // CH
ROWS_PER_SUB = NROWS // 16
NP = 10240
NODES_PER_S = NP // 16
ZCH = 128


def _sc_aggregate(dh, h2, srcm, dstm):
    mesh = plsc.VectorSubcoreMesh(core_axis_name="c", subcore_axis_name="s")

    @functools.partial(
        pl.kernel,
        out_type=jax.ShapeDtypeStruct((2, NP, dh), jnp.float32),
        mesh=mesh,
        compiler_params=pltpu.CompilerParams(use_tc_tiling_on_sc=False),
        scratch_types=[
            pltpu.VMEM((ROWS_PER_SUB, CH), jnp.int32),
            pltpu.VMEM((ROWS_PER_SUB, CH), jnp.int32),
            pltpu.VMEM((CH, dh), jnp.float32),
            pltpu.VMEM((CH, dh), jnp.float32),
            pltpu.VMEM((ZCH, dh), jnp.float32),
            pltpu.VMEM_SHARED((NP, dh), jnp.float32),
            pltpu.SemaphoreType.DMA,
            pltpu.SemaphoreType.DMA,
        ],
    )
    def agg_kernel(h_hbm, src_hbm, dst_hbm, out_hbm,
                   sidx, didx, rows_a, rows_b, zbuf, acc, gs_a, gs_b):
        cid = lax.axis_index("c")
        sid = lax.axis_index("s")
        row0 = sid * ROWS_PER_SUB

        pltpu.sync_copy(src_hbm.at[pl.ds(row0, ROWS_PER_SUB)], sidx)
        pltpu.sync_copy(dst_hbm.at[pl.ds(row0, ROWS_PER_SUB)], didx)

        zv = jnp.zeros((16,), jnp.float32)

        @pl.loop(0, ZCH)
        def _(i):
            @pl.loop(0, dh, step=16)
            def _(j):
                zbuf[i, pl.ds(j, 16)] = zv

        @pl.loop(0, NODES_PER_S // ZCH)
        def _(k):
            pltpu.sync_copy(zbuf, acc.at[pl.ds(sid * NODES_PER_S + k * ZCH, ZCH)])

        plsc.subcore_barrier()

        h_slab = h_hbm.at[cid]
        pltpu.async_copy(h_slab.at[sidx.at[0]], rows_a, gs_a)

        @pl.loop(0, ROWS_PER_SUB, step=2)
        def _(j):
            pltpu.make_async_copy(h_slab.at[sidx.at[j]], rows_a, gs_a).wait()
            pltpu.async_copy(h_slab.at[sidx.at[j + 1]], rows_b, gs_b)
            pltpu.sync_copy(rows_a, acc.at[didx.at[j]], add=True)
            pltpu.make_async_copy(h_slab.at[sidx.at[j]], rows_b, gs_b).wait()

            @pl.when(j + 2 < ROWS_PER_SUB)
            def _():
                pltpu.async_copy(h_slab.at[sidx.at[j + 2]], rows_a, gs_a)

            pltpu.sync_copy(rows_b, acc.at[didx.at[j + 1]], add=True)

        plsc.subcore_barrier()

        pltpu.sync_copy(acc.at[pl.ds(sid * NODES_PER_S, NODES_PER_S)],
                        out_hbm.at[cid].at[pl.ds(sid * NODES_PER_S, NODES_PER_S)])

    return agg_kernel(h2, srcm, dstm)


def _split_halves(r):
    dh = r.shape[1] // 2
    return jnp.stack([r[:, :dh], r[:, dh:]])


def _mm_body(x_ref, w_ref, o_ref):
    r = jnp.dot(x_ref[...], w_ref[...], preferred_element_type=jnp.float32)
    o_ref[...] = _split_halves(r)


def _tc_matmul(x, w):
    return pl.pallas_call(
        _mm_body,
        out_shape=jax.ShapeDtypeStruct((2, x.shape[0], w.shape[1] // 2),
                                       jnp.float32),
    )(x, w)


def _mid_body(a_ref, b_ref, g_ref, be_ref, w_ref, o_ref):
    h = jnp.concatenate([a_ref[0, :N], a_ref[1, :N]], axis=1) + b_ref[...]
    mu = jnp.mean(h, axis=0, keepdims=True)
    var = jnp.mean((h - mu) ** 2, axis=0, keepdims=True)
    hn = (h - mu) / jnp.sqrt(var + 1e-5) * g_ref[...] + be_ref[...]
    hn = jnp.maximum(hn, 0.0)
    r = jnp.dot(hn, w_ref[...], preferred_element_type=jnp.float32)
    o_ref[...] = _split_halves(r)


def _tc_mid(a, b, g, be, w):
    return pl.pallas_call(
        _mid_body,
        out_shape=jax.ShapeDtypeStruct((2, N, w.shape[1] // 2), jnp.float32),
    )(a, b.reshape(1, -1), g.reshape(1, -1), be.reshape(1, -1), w)


def _fin_body(a_ref, b_ref, o_ref):
    h = jnp.concatenate([a_ref[0, :N], a_ref[1, :N]], axis=1) + b_ref[...]
    col = lax.broadcasted_iota(jnp.int32, h.shape, 1)
    valid = col < D_OUT
    m = jnp.max(jnp.where(valid, h, -jnp.inf), axis=1, keepdims=True)
    ex = jnp.where(valid, jnp.exp(h - m), 0.0)
    lse = m + jnp.log(jnp.sum(ex, axis=1, keepdims=True))
    o_ref[...] = h - lse


def _tc_final(a, b):
    return pl.pallas_call(
        _fin_body,
        out_shape=jax.ShapeDtypeStruct((N, D_PAD), jnp.float32),
    )(a, b.reshape(1, -1))


def kernel(x, edge_index, W0, b0, g0, be0, W1, b1, g1, be1, W2, b2):
    srcm = edge_index[0].reshape(NROWS, CH)
    dstm = edge_index[1].reshape(NROWS, CH)
    W2p = jnp.pad(W2, ((0, 0), (0, D_PAD - D_OUT)))
    b2p = jnp.pad(b2, (0, D_PAD - D_OUT))

    h0 = _tc_matmul(x, W0)
    a0 = _sc_aggregate(64, h0, srcm, dstm)
    h1 = _tc_mid(a0, b0, g0, be0, W1)
    a1 = _sc_aggregate(64, h1, srcm, dstm)
    h2 = _tc_mid(a1, b1, g1, be1, W2p)
    a2 = _sc_aggregate(32, h2, srcm, dstm)
    out = _tc_final(a2, b2p)
    return out[:, :D_OUT]
